# baseline (speedup 1.0000x reference)
import functools

import jax
import jax.numpy as jnp
from jax import lax
from jax.experimental import pallas as pl
from jax.experimental.pallas import tpu as pltpu
from jax.experimental.pallas import tpu_sc as plsc

N = 10000
D = 128
EDGES = 320000
NPRED = 100000

NC = 2
NS = 16
NW = NC * NS

E_CH = 1250
E_CE = 8
NPAD = 10240
ROWS_PER_TILE = NPAD // NS

P_PAD = 100352
P_CH = 28
P_CP = 112

_mesh = functools.partial(
    plsc.VectorSubcoreMesh, core_axis_name="c", subcore_axis_name="s")


EW = EDGES // NW


def _deg_body(dstW, deg_out, idx_v, hist):
  c = lax.axis_index("c")
  s = lax.axis_index("s")
  w = c * NS + s
  pltpu.sync_copy(dstW.at[w, 0], idx_v)
  zeros = jnp.zeros((16,), jnp.float32)

  def z(i, carry):
    hist[pl.ds(i * 16, 16)] = zeros
    return carry

  lax.fori_loop(0, NPAD // 16, z, 0)
  ones = jnp.ones((16,), jnp.float32)

  def acc(i, carry):
    idx16 = idx_v[pl.ds(i * 16, 16)]
    plsc.addupdate_scatter(hist, [idx16], ones)
    return carry

  lax.fori_loop(0, EW // 16, acc, 0)
  pltpu.sync_copy(hist, deg_out.at[w, 0])


def _sc_degree(dstW):
  return pl.kernel(
      _deg_body,
      out_type=jax.ShapeDtypeStruct((NW, 1, NPAD), jnp.float32),
      mesh=_mesh(),
      compiler_params=pltpu.CompilerParams(needs_layout_passes=False),
      scratch_types=[
          pltpu.VMEM((EW,), jnp.int32),
          pltpu.VMEM((NPAD,), jnp.float32),
      ],
  )(dstW)


def _agg_body(y_hbm, srcE, dstE, z128, agg_out, src_v, dst_v, r_a, r_b, shared,
              sems):
  sem_a = sems.at[0]
  sem_b = sems.at[1]
  c = lax.axis_index("c")
  s = lax.axis_index("s")
  pltpu.sync_copy(srcE.at[c, s], src_v)
  pltpu.sync_copy(dstE.at[c, s], dst_v)
  for z in range(ROWS_PER_TILE // 128):
    pltpu.sync_copy(z128, shared.at[pl.ds(s * ROWS_PER_TILE + z * 128, 128)])
  plsc.subcore_barrier()

  def sidx(j):
    return src_v.at[pl.ds(j * E_CE, E_CE)]

  def didx(j):
    return dst_v.at[pl.ds(j * E_CE, E_CE)]

  pltpu.async_copy(y_hbm.at[sidx(0)], r_a, sem_a)
  nh = E_CH // 2

  def body(i, carry):
    j0 = 2 * i
    pltpu.make_async_copy(y_hbm.at[sidx(j0)], r_a, sem_a).wait()
    pltpu.async_copy(y_hbm.at[sidx(j0 + 1)], r_b, sem_b)
    pltpu.sync_copy(r_a, shared.at[didx(j0)], add=True)
    pltpu.make_async_copy(y_hbm.at[sidx(j0 + 1)], r_b, sem_b).wait()

    @pl.when(i < nh - 1)
    def _():
      pltpu.async_copy(y_hbm.at[sidx(j0 + 2)], r_a, sem_a)

    pltpu.sync_copy(r_b, shared.at[didx(j0 + 1)], add=True)
    return carry

  lax.fori_loop(0, nh, body, 0)
  plsc.subcore_barrier()
  pltpu.sync_copy(
      shared.at[pl.ds(s * ROWS_PER_TILE, ROWS_PER_TILE)],
      agg_out.at[c, pl.ds(s * ROWS_PER_TILE, ROWS_PER_TILE)])


def _sc_agg(y, srcE, dstE, z128):
  return pl.kernel(
      _agg_body,
      out_type=jax.ShapeDtypeStruct((NC, NPAD, D), jnp.float32),
      mesh=_mesh(),
      scratch_types=[
          pltpu.VMEM((E_CH * E_CE,), jnp.int32),
          pltpu.VMEM((E_CH * E_CE,), jnp.int32),
          pltpu.VMEM((E_CE, D), jnp.float32),
          pltpu.VMEM((E_CE, D), jnp.float32),
          pltpu.VMEM_SHARED((NPAD, D), jnp.float32),
          pltpu.SemaphoreType.DMA((2,)),
      ],
  )(y, srcE, dstE, z128)


def _score_body(h_hbm, e0, e1, out, i0_v, i1_v, ra0, rb0, ra1, rb1, scores_v,
                sem_a, sem_b):
  c = lax.axis_index("c")
  s = lax.axis_index("s")
  pltpu.sync_copy(e0.at[c, s], i0_v)
  pltpu.sync_copy(e1.at[c, s], i1_v)
  lanes = lax.iota(jnp.int32, 16)
  wbase = (c * NS + s) * P_CH * P_CP

  def gather(j, buf_a, buf_b, sem):
    pltpu.async_copy(h_hbm.at[i0_v.at[j]], buf_a, sem)
    pltpu.async_copy(h_hbm.at[i1_v.at[j]], buf_b, sem)

  def gwait(j, buf_a, buf_b, sem):
    pltpu.make_async_copy(h_hbm.at[i0_v.at[j]], buf_a, sem).wait()
    pltpu.make_async_copy(h_hbm.at[i1_v.at[j]], buf_b, sem).wait()

  def compute(j, buf_a, buf_b):
    for g in range(P_CP // 16):
      pid = lanes + (g * 16)

      def kblock(t, acc):
        for u in range(8):
          kk = jnp.full((16,), t * 8 + u, dtype=jnp.int32)
          a = plsc.load_gather(buf_a, [pid, kk])
          b = plsc.load_gather(buf_b, [pid, kk])
          acc = acc + a * b
        return acc

      dot = lax.fori_loop(0, D // 8, kblock, jnp.zeros((16,), jnp.float32))
      scores_v[pl.ds(g * 16, 16)] = 1.0 / (1.0 + jnp.exp(-dot))
    pltpu.sync_copy(scores_v, out.at[pl.ds(wbase + j * P_CP, P_CP)])

  gather(0, ra0, rb0, sem_a)
  nh = P_CH // 2

  def body(i, carry):
    j0 = 2 * i
    gwait(j0, ra0, rb0, sem_a)
    gather(j0 + 1, ra1, rb1, sem_b)
    compute(j0, ra0, rb0)
    gwait(j0 + 1, ra1, rb1, sem_b)

    @pl.when(i < nh - 1)
    def _():
      gather(j0 + 2, ra0, rb0, sem_a)

    compute(j0 + 1, ra1, rb1)
    return carry

  lax.fori_loop(0, nh, body, 0)


def _sc_score(h2, e0, e1):
  return pl.kernel(
      _score_body,
      out_type=jax.ShapeDtypeStruct((P_PAD,), jnp.float32),
      mesh=_mesh(),
      compiler_params=pltpu.CompilerParams(needs_layout_passes=False),
      scratch_types=[
          pltpu.VMEM((P_CH, P_CP), jnp.int32),
          pltpu.VMEM((P_CH, P_CP), jnp.int32),
          pltpu.VMEM((P_CP, D), jnp.float32),
          pltpu.VMEM((P_CP, D), jnp.float32),
          pltpu.VMEM((P_CP, D), jnp.float32),
          pltpu.VMEM((P_CP, D), jnp.float32),
          pltpu.VMEM((P_CP,), jnp.float32),
          pltpu.SemaphoreType.DMA,
          pltpu.SemaphoreType.DMA,
      ],
  )(h2, e0, e1)


_RB = 1024


def _tc1_body(deg_ref, x_ref, w_ref, y_ref, dinv_ref):
  ones_w = jnp.ones((NW, 1), jnp.float32)
  deg = lax.dot_general(
      deg_ref[...], ones_w, (((0,), (0,)), ((), ())),
      preferred_element_type=jnp.float32) + 1.0
  dinv = lax.rsqrt(deg)
  xw = jnp.dot(x_ref[...], w_ref[...], preferred_element_type=jnp.float32)
  y_ref[...] = xw * dinv
  dinv_ref[...] = dinv


def _tc1(deg_p, x, w1):
  return pl.pallas_call(
      _tc1_body,
      grid=(NPAD // _RB,),
      in_specs=[
          pl.BlockSpec((NW, _RB), lambda i: (0, i)),
          pl.BlockSpec((_RB, D), lambda i: (i, 0)),
          pl.BlockSpec((D, D), lambda i: (0, 0)),
      ],
      out_specs=[
          pl.BlockSpec((_RB, D), lambda i: (i, 0)),
          pl.BlockSpec((_RB, 1), lambda i: (i, 0)),
      ],
      out_shape=[
          jax.ShapeDtypeStruct((NPAD, D), jnp.float32),
          jax.ShapeDtypeStruct((NPAD, 1), jnp.float32),
      ],
  )(deg_p, x, w1)


def _tc2_body(agg_ref, y1_ref, dinv_ref, b1_ref, w2_ref, y2_ref):
  tot = agg_ref[0] + agg_ref[1] + y1_ref[...]
  h = jnp.maximum(tot * dinv_ref[...] + b1_ref[...], 0.0)
  y2_ref[...] = jnp.dot(
      h, w2_ref[...], preferred_element_type=jnp.float32) * dinv_ref[...]


def _tc2(agg1, y1, dinv, b1, w2):
  return pl.pallas_call(
      _tc2_body,
      grid=(NPAD // _RB,),
      in_specs=[
          pl.BlockSpec((NC, _RB, D), lambda i: (0, i, 0)),
          pl.BlockSpec((_RB, D), lambda i: (i, 0)),
          pl.BlockSpec((_RB, 1), lambda i: (i, 0)),
          pl.BlockSpec((1, D), lambda i: (0, 0)),
          pl.BlockSpec((D, D), lambda i: (0, 0)),
      ],
      out_specs=pl.BlockSpec((_RB, D), lambda i: (i, 0)),
      out_shape=jax.ShapeDtypeStruct((NPAD, D), jnp.float32),
  )(agg1, y1, dinv, b1, w2)


def _tc3_body(agg_ref, y2_ref, dinv_ref, b2_ref, h2_ref):
  tot = agg_ref[0] + agg_ref[1] + y2_ref[...]
  h2_ref[...] = tot * dinv_ref[...] + b2_ref[...]


def _tc3(agg2, y2, dinv, b2):
  return pl.pallas_call(
      _tc3_body,
      grid=(NPAD // _RB,),
      in_specs=[
          pl.BlockSpec((NC, _RB, D), lambda i: (0, i, 0)),
          pl.BlockSpec((_RB, D), lambda i: (i, 0)),
          pl.BlockSpec((_RB, 1), lambda i: (i, 0)),
          pl.BlockSpec((1, D), lambda i: (0, 0)),
      ],
      out_specs=pl.BlockSpec((_RB, D), lambda i: (i, 0)),
      out_shape=jax.ShapeDtypeStruct((NPAD, D), jnp.float32),
  )(agg2, y2, dinv, b2)


def kernel(Features, A, E, W1, b1, W2, b2):
  srcE = A[0].reshape(NC, NS, E_CH * E_CE)
  dstE = A[1].reshape(NC, NS, E_CH * E_CE)
  dstW = A[1].reshape(NW, 1, EW)
  epad = jnp.concatenate(
      [E, jnp.zeros((2, P_PAD - NPRED), dtype=E.dtype)], axis=1)
  e0 = epad[0].reshape(NC, NS, P_CH, P_CP)
  e1 = epad[1].reshape(NC, NS, P_CH, P_CP)
  z128 = jnp.zeros((128, D), jnp.float32)
  b1r = b1.reshape(1, D)
  b2r = b2.reshape(1, D)

  deg_p = _sc_degree(dstW).reshape(NW, NPAD)
  y1, dinv = _tc1(deg_p, Features, W1)
  agg1 = _sc_agg(y1, srcE, dstE, z128)
  y2 = _tc2(agg1, y1, dinv, b1r, W2)
  agg2 = _sc_agg(y2, srcE, dstE, z128)
  h2 = _tc3(agg2, y2, dinv, b2r)
  scores = _sc_score(h2, e0, e1)
  return scores[:NPRED]


# speedup vs baseline: 2.4944x; 2.4944x over previous
import functools

import jax
import jax.numpy as jnp
from jax import lax
from jax.experimental import pallas as pl
from jax.experimental.pallas import tpu as pltpu
from jax.experimental.pallas import tpu_sc as plsc

N = 10000
D = 128
EDGES = 320000
NPRED = 100000

NC = 2
NS = 16
NW = NC * NS

E_CH = 125
E_CE = 80
SCAT_R = 8
NPAD = 10240
ROWS_PER_TILE = NPAD // NS

P_PAD = 100352
P_CH = 28
P_CP = 112

_mesh = functools.partial(
    plsc.VectorSubcoreMesh, core_axis_name="c", subcore_axis_name="s")


EW = EDGES // NW


def _deg_body(dstW, deg_out, idx_v, hist):
  c = lax.axis_index("c")
  s = lax.axis_index("s")
  w = c * NS + s
  pltpu.sync_copy(dstW.at[w, 0], idx_v)
  zeros = jnp.zeros((16,), jnp.float32)

  def z(i, carry):
    hist[pl.ds(i * 16, 16)] = zeros
    return carry

  lax.fori_loop(0, NPAD // 16, z, 0)
  ones = jnp.ones((16,), jnp.float32)

  def acc(i, carry):
    idx16 = idx_v[pl.ds(i * 16, 16)]
    plsc.addupdate_scatter(hist, [idx16], ones)
    return carry

  lax.fori_loop(0, EW // 16, acc, 0)
  pltpu.sync_copy(hist, deg_out.at[w, 0])


def _sc_degree(dstW):
  return pl.kernel(
      _deg_body,
      out_type=jax.ShapeDtypeStruct((NW, 1, NPAD), jnp.float32),
      mesh=_mesh(),
      compiler_params=pltpu.CompilerParams(needs_layout_passes=False),
      scratch_types=[
          pltpu.VMEM((EW,), jnp.int32),
          pltpu.VMEM((NPAD,), jnp.float32),
      ],
  )(dstW)


def _agg_body(y_hbm, srcE, dstE, z128, agg_out, src_v, dst_v, r_a, r_b, shared,
              sems):
  sem_a = sems.at[0]
  sem_b = sems.at[1]
  sem_sa = sems.at[2]
  sem_sb = sems.at[3]
  c = lax.axis_index("c")
  s = lax.axis_index("s")
  pltpu.sync_copy(srcE.at[c, s], src_v)
  pltpu.sync_copy(dstE.at[c, s], dst_v)
  for z in range(ROWS_PER_TILE // 128):
    pltpu.sync_copy(z128, shared.at[pl.ds(s * ROWS_PER_TILE + z * 128, 128)])
  plsc.subcore_barrier()

  def sidx(j):
    return src_v.at[pl.ds(j * E_CE, E_CE)]

  def scat_issue(buf, j, sem):
    for k in range(E_CE // SCAT_R):
      idx8 = dst_v.at[pl.ds(j * E_CE + k * SCAT_R, SCAT_R)]
      pltpu.async_copy(
          buf.at[pl.ds(k * SCAT_R, SCAT_R)], shared.at[idx8], sem, add=True)

  def scat_drain(buf, j, sem):
    for k in range(E_CE // SCAT_R):
      idx8 = dst_v.at[pl.ds(j * E_CE + k * SCAT_R, SCAT_R)]
      pltpu.make_async_copy(
          buf.at[pl.ds(k * SCAT_R, SCAT_R)], shared.at[idx8], sem).wait()

  pltpu.async_copy(y_hbm.at[sidx(0)], r_a, sem_a)
  nh = E_CH // 2

  def body(i, carry):
    j0 = 2 * i
    pltpu.make_async_copy(y_hbm.at[sidx(j0)], r_a, sem_a).wait()
    pltpu.async_copy(y_hbm.at[sidx(j0 + 1)], r_b, sem_b)
    scat_issue(r_a, j0, sem_sa)
    pltpu.make_async_copy(y_hbm.at[sidx(j0 + 1)], r_b, sem_b).wait()
    scat_drain(r_a, j0, sem_sa)

    @pl.when(i < nh - 1)
    def _():
      pltpu.async_copy(y_hbm.at[sidx(j0 + 2)], r_a, sem_a)

    scat_issue(r_b, j0 + 1, sem_sb)
    scat_drain(r_b, j0 + 1, sem_sb)
    return carry

  lax.fori_loop(0, nh, body, 0)
  plsc.subcore_barrier()
  pltpu.sync_copy(
      shared.at[pl.ds(s * ROWS_PER_TILE, ROWS_PER_TILE)],
      agg_out.at[c, pl.ds(s * ROWS_PER_TILE, ROWS_PER_TILE)])


def _sc_agg(y, srcE, dstE, z128):
  return pl.kernel(
      _agg_body,
      out_type=jax.ShapeDtypeStruct((NC, NPAD, D), jnp.float32),
      mesh=_mesh(),
      scratch_types=[
          pltpu.VMEM((E_CH * E_CE,), jnp.int32),
          pltpu.VMEM((E_CH * E_CE,), jnp.int32),
          pltpu.VMEM((E_CE, D), jnp.float32),
          pltpu.VMEM((E_CE, D), jnp.float32),
          pltpu.VMEM_SHARED((NPAD, D), jnp.float32),
          pltpu.SemaphoreType.DMA((4,)),
      ],
  )(y, srcE, dstE, z128)


def _score_body(h_hbm, e0, e1, out, i0_v, i1_v, ra0, rb0, ra1, rb1, scores_v,
                sem_a, sem_b):
  c = lax.axis_index("c")
  s = lax.axis_index("s")
  pltpu.sync_copy(e0.at[c, s], i0_v)
  pltpu.sync_copy(e1.at[c, s], i1_v)
  lanes = lax.iota(jnp.int32, 16)
  wbase = (c * NS + s) * P_CH * P_CP

  def gather(j, buf_a, buf_b, sem):
    pltpu.async_copy(h_hbm.at[i0_v.at[j]], buf_a, sem)
    pltpu.async_copy(h_hbm.at[i1_v.at[j]], buf_b, sem)

  def gwait(j, buf_a, buf_b, sem):
    pltpu.make_async_copy(h_hbm.at[i0_v.at[j]], buf_a, sem).wait()
    pltpu.make_async_copy(h_hbm.at[i1_v.at[j]], buf_b, sem).wait()

  def compute(j, buf_a, buf_b):
    for g in range(P_CP // 16):
      pid = lanes + (g * 16)

      def kblock(t, acc):
        for u in range(8):
          kk = jnp.full((16,), t * 8 + u, dtype=jnp.int32)
          a = plsc.load_gather(buf_a, [pid, kk])
          b = plsc.load_gather(buf_b, [pid, kk])
          acc = acc + a * b
        return acc

      dot = lax.fori_loop(0, D // 8, kblock, jnp.zeros((16,), jnp.float32))
      scores_v[pl.ds(g * 16, 16)] = 1.0 / (1.0 + jnp.exp(-dot))
    pltpu.sync_copy(scores_v, out.at[pl.ds(wbase + j * P_CP, P_CP)])

  gather(0, ra0, rb0, sem_a)
  nh = P_CH // 2

  def body(i, carry):
    j0 = 2 * i
    gwait(j0, ra0, rb0, sem_a)
    gather(j0 + 1, ra1, rb1, sem_b)
    compute(j0, ra0, rb0)
    gwait(j0 + 1, ra1, rb1, sem_b)

    @pl.when(i < nh - 1)
    def _():
      gather(j0 + 2, ra0, rb0, sem_a)

    compute(j0 + 1, ra1, rb1)
    return carry

  lax.fori_loop(0, nh, body, 0)


def _sc_score(h2, e0, e1):
  return pl.kernel(
      _score_body,
      out_type=jax.ShapeDtypeStruct((P_PAD,), jnp.float32),
      mesh=_mesh(),
      compiler_params=pltpu.CompilerParams(needs_layout_passes=False),
      scratch_types=[
          pltpu.VMEM((P_CH, P_CP), jnp.int32),
          pltpu.VMEM((P_CH, P_CP), jnp.int32),
          pltpu.VMEM((P_CP, D), jnp.float32),
          pltpu.VMEM((P_CP, D), jnp.float32),
          pltpu.VMEM((P_CP, D), jnp.float32),
          pltpu.VMEM((P_CP, D), jnp.float32),
          pltpu.VMEM((P_CP,), jnp.float32),
          pltpu.SemaphoreType.DMA,
          pltpu.SemaphoreType.DMA,
      ],
  )(h2, e0, e1)


_RB = 1024


def _tc1_body(deg_ref, x_ref, w_ref, y_ref, dinv_ref):
  ones_w = jnp.ones((NW, 1), jnp.float32)
  deg = lax.dot_general(
      deg_ref[...], ones_w, (((0,), (0,)), ((), ())),
      preferred_element_type=jnp.float32) + 1.0
  dinv = lax.rsqrt(deg)
  xw = jnp.dot(x_ref[...], w_ref[...], preferred_element_type=jnp.float32)
  y_ref[...] = xw * dinv
  dinv_ref[...] = dinv


def _tc1(deg_p, x, w1):
  return pl.pallas_call(
      _tc1_body,
      grid=(NPAD // _RB,),
      in_specs=[
          pl.BlockSpec((NW, _RB), lambda i: (0, i)),
          pl.BlockSpec((_RB, D), lambda i: (i, 0)),
          pl.BlockSpec((D, D), lambda i: (0, 0)),
      ],
      out_specs=[
          pl.BlockSpec((_RB, D), lambda i: (i, 0)),
          pl.BlockSpec((_RB, 1), lambda i: (i, 0)),
      ],
      out_shape=[
          jax.ShapeDtypeStruct((NPAD, D), jnp.float32),
          jax.ShapeDtypeStruct((NPAD, 1), jnp.float32),
      ],
  )(deg_p, x, w1)


def _tc2_body(agg_ref, y1_ref, dinv_ref, b1_ref, w2_ref, y2_ref):
  tot = agg_ref[0] + agg_ref[1] + y1_ref[...]
  h = jnp.maximum(tot * dinv_ref[...] + b1_ref[...], 0.0)
  y2_ref[...] = jnp.dot(
      h, w2_ref[...], preferred_element_type=jnp.float32) * dinv_ref[...]


def _tc2(agg1, y1, dinv, b1, w2):
  return pl.pallas_call(
      _tc2_body,
      grid=(NPAD // _RB,),
      in_specs=[
          pl.BlockSpec((NC, _RB, D), lambda i: (0, i, 0)),
          pl.BlockSpec((_RB, D), lambda i: (i, 0)),
          pl.BlockSpec((_RB, 1), lambda i: (i, 0)),
          pl.BlockSpec((1, D), lambda i: (0, 0)),
          pl.BlockSpec((D, D), lambda i: (0, 0)),
      ],
      out_specs=pl.BlockSpec((_RB, D), lambda i: (i, 0)),
      out_shape=jax.ShapeDtypeStruct((NPAD, D), jnp.float32),
  )(agg1, y1, dinv, b1, w2)


def _tc3_body(agg_ref, y2_ref, dinv_ref, b2_ref, h2_ref):
  tot = agg_ref[0] + agg_ref[1] + y2_ref[...]
  h2_ref[...] = tot * dinv_ref[...] + b2_ref[...]


def _tc3(agg2, y2, dinv, b2):
  return pl.pallas_call(
      _tc3_body,
      grid=(NPAD // _RB,),
      in_specs=[
          pl.BlockSpec((NC, _RB, D), lambda i: (0, i, 0)),
          pl.BlockSpec((_RB, D), lambda i: (i, 0)),
          pl.BlockSpec((_RB, 1), lambda i: (i, 0)),
          pl.BlockSpec((1, D), lambda i: (0, 0)),
      ],
      out_specs=pl.BlockSpec((_RB, D), lambda i: (i, 0)),
      out_shape=jax.ShapeDtypeStruct((NPAD, D), jnp.float32),
  )(agg2, y2, dinv, b2)


def kernel(Features, A, E, W1, b1, W2, b2):
  srcE = A[0].reshape(NC, NS, E_CH * E_CE)
  dstE = A[1].reshape(NC, NS, E_CH * E_CE)
  dstW = A[1].reshape(NW, 1, EW)
  epad = jnp.concatenate(
      [E, jnp.zeros((2, P_PAD - NPRED), dtype=E.dtype)], axis=1)
  e0 = epad[0].reshape(NC, NS, P_CH, P_CP)
  e1 = epad[1].reshape(NC, NS, P_CH, P_CP)
  z128 = jnp.zeros((128, D), jnp.float32)
  b1r = b1.reshape(1, D)
  b2r = b2.reshape(1, D)

  deg_p = _sc_degree(dstW).reshape(NW, NPAD)
  y1, dinv = _tc1(deg_p, Features, W1)
  agg1 = _sc_agg(y1, srcE, dstE, z128)
  y2 = _tc2(agg1, y1, dinv, b1r, W2)
  agg2 = _sc_agg(y2, srcE, dstE, z128)
  h2 = _tc3(agg2, y2, dinv, b2r)
  scores = _sc_score(h2, e0, e1)
  return scores[:NPRED]


# speedup vs baseline: 2.4962x; 1.0007x over previous
import functools

import jax
import jax.numpy as jnp
from jax import lax
from jax.experimental import pallas as pl
from jax.experimental.pallas import tpu as pltpu
from jax.experimental.pallas import tpu_sc as plsc

N = 10000
D = 128
EDGES = 320000
NPRED = 100000

NC = 2
NS = 16
NW = NC * NS

E_CH = 125
E_CE = 80
SCAT_R = 8
NPAD = 10240
ROWS_PER_TILE = NPAD // NS

P_PAD = 100352
P_CH = 28
P_CP = 112

_mesh = functools.partial(
    plsc.VectorSubcoreMesh, core_axis_name="c", subcore_axis_name="s")


EW = EDGES // NW


def _deg_body(dstW, deg_out, idx_v, hist):
  c = lax.axis_index("c")
  s = lax.axis_index("s")
  w = c * NS + s
  pltpu.sync_copy(dstW.at[w, 0], idx_v)
  zeros = jnp.zeros((16,), jnp.float32)

  def z(i, carry):
    hist[pl.ds(i * 16, 16)] = zeros
    return carry

  lax.fori_loop(0, NPAD // 16, z, 0)
  ones = jnp.ones((16,), jnp.float32)

  def acc(i, carry):
    idx16 = idx_v[pl.ds(i * 16, 16)]
    plsc.addupdate_scatter(hist, [idx16], ones)
    return carry

  lax.fori_loop(0, EW // 16, acc, 0)
  pltpu.sync_copy(hist, deg_out.at[w, 0])


def _sc_degree(dstW):
  return pl.kernel(
      _deg_body,
      out_type=jax.ShapeDtypeStruct((NW, 1, NPAD), jnp.float32),
      mesh=_mesh(),
      compiler_params=pltpu.CompilerParams(needs_layout_passes=False),
      scratch_types=[
          pltpu.VMEM((EW,), jnp.int32),
          pltpu.VMEM((NPAD,), jnp.float32),
      ],
  )(dstW)


def _agg_body(y_hbm, srcE, dstE, z128, agg_out, src_v, dst_v, r_a, r_b, shared,
              sems):
  sem_a = sems.at[0]
  sem_b = sems.at[1]
  c = lax.axis_index("c")
  s = lax.axis_index("s")
  pltpu.sync_copy(srcE.at[c, s], src_v)
  pltpu.sync_copy(dstE.at[c, s], dst_v)
  for z in range(ROWS_PER_TILE // 128):
    pltpu.sync_copy(z128, shared.at[pl.ds(s * ROWS_PER_TILE + z * 128, 128)])
  plsc.subcore_barrier()

  def sidx(j):
    return src_v.at[pl.ds(j * E_CE, E_CE)]

  def scat_sync(buf, j):
    for k in range(E_CE // SCAT_R):
      idx8 = dst_v.at[pl.ds(j * E_CE + k * SCAT_R, SCAT_R)]
      pltpu.sync_copy(
          buf.at[pl.ds(k * SCAT_R, SCAT_R)], shared.at[idx8], add=True)

  pltpu.async_copy(y_hbm.at[sidx(0)], r_a, sem_a)
  nh = E_CH // 2

  def body(i, carry):
    j0 = 2 * i
    pltpu.make_async_copy(y_hbm.at[sidx(j0)], r_a, sem_a).wait()
    pltpu.async_copy(y_hbm.at[sidx(j0 + 1)], r_b, sem_b)
    scat_sync(r_a, j0)
    pltpu.make_async_copy(y_hbm.at[sidx(j0 + 1)], r_b, sem_b).wait()

    @pl.when(i < nh - 1)
    def _():
      pltpu.async_copy(y_hbm.at[sidx(j0 + 2)], r_a, sem_a)

    scat_sync(r_b, j0 + 1)
    return carry

  lax.fori_loop(0, nh, body, 0)
  plsc.subcore_barrier()
  pltpu.sync_copy(
      shared.at[pl.ds(s * ROWS_PER_TILE, ROWS_PER_TILE)],
      agg_out.at[c, pl.ds(s * ROWS_PER_TILE, ROWS_PER_TILE)])


def _sc_agg(y, srcE, dstE, z128):
  return pl.kernel(
      _agg_body,
      out_type=jax.ShapeDtypeStruct((NC, NPAD, D), jnp.float32),
      mesh=_mesh(),
      scratch_types=[
          pltpu.VMEM((E_CH * E_CE,), jnp.int32),
          pltpu.VMEM((E_CH * E_CE,), jnp.int32),
          pltpu.VMEM((E_CE, D), jnp.float32),
          pltpu.VMEM((E_CE, D), jnp.float32),
          pltpu.VMEM_SHARED((NPAD, D), jnp.float32),
          pltpu.SemaphoreType.DMA((2,)),
      ],
  )(y, srcE, dstE, z128)


def _score_body(h_hbm, e0, e1, out, i0_v, i1_v, ra0, rb0, ra1, rb1, scores_v,
                sem_a, sem_b):
  c = lax.axis_index("c")
  s = lax.axis_index("s")
  pltpu.sync_copy(e0.at[c, s], i0_v)
  pltpu.sync_copy(e1.at[c, s], i1_v)
  lanes = lax.iota(jnp.int32, 16)
  wbase = (c * NS + s) * P_CH * P_CP

  def gather(j, buf_a, buf_b, sem):
    pltpu.async_copy(h_hbm.at[i0_v.at[j]], buf_a, sem)
    pltpu.async_copy(h_hbm.at[i1_v.at[j]], buf_b, sem)

  def gwait(j, buf_a, buf_b, sem):
    pltpu.make_async_copy(h_hbm.at[i0_v.at[j]], buf_a, sem).wait()
    pltpu.make_async_copy(h_hbm.at[i1_v.at[j]], buf_b, sem).wait()

  def compute(j, buf_a, buf_b):
    for g in range(P_CP // 16):
      pid = lanes + (g * 16)

      def kblock(t, acc):
        for u in range(8):
          kk = jnp.full((16,), t * 8 + u, dtype=jnp.int32)
          a = plsc.load_gather(buf_a, [pid, kk])
          b = plsc.load_gather(buf_b, [pid, kk])
          acc = acc + a * b
        return acc

      dot = lax.fori_loop(0, D // 8, kblock, jnp.zeros((16,), jnp.float32))
      scores_v[pl.ds(g * 16, 16)] = 1.0 / (1.0 + jnp.exp(-dot))
    pltpu.sync_copy(scores_v, out.at[pl.ds(wbase + j * P_CP, P_CP)])

  gather(0, ra0, rb0, sem_a)
  nh = P_CH // 2

  def body(i, carry):
    j0 = 2 * i
    gwait(j0, ra0, rb0, sem_a)
    gather(j0 + 1, ra1, rb1, sem_b)
    compute(j0, ra0, rb0)
    gwait(j0 + 1, ra1, rb1, sem_b)

    @pl.when(i < nh - 1)
    def _():
      gather(j0 + 2, ra0, rb0, sem_a)

    compute(j0 + 1, ra1, rb1)
    return carry

  lax.fori_loop(0, nh, body, 0)


def _sc_score(h2, e0, e1):
  return pl.kernel(
      _score_body,
      out_type=jax.ShapeDtypeStruct((P_PAD,), jnp.float32),
      mesh=_mesh(),
      compiler_params=pltpu.CompilerParams(needs_layout_passes=False),
      scratch_types=[
          pltpu.VMEM((P_CH, P_CP), jnp.int32),
          pltpu.VMEM((P_CH, P_CP), jnp.int32),
          pltpu.VMEM((P_CP, D), jnp.float32),
          pltpu.VMEM((P_CP, D), jnp.float32),
          pltpu.VMEM((P_CP, D), jnp.float32),
          pltpu.VMEM((P_CP, D), jnp.float32),
          pltpu.VMEM((P_CP,), jnp.float32),
          pltpu.SemaphoreType.DMA,
          pltpu.SemaphoreType.DMA,
      ],
  )(h2, e0, e1)


_RB = 1024


def _tc1_body(deg_ref, x_ref, w_ref, y_ref, dinv_ref):
  ones_w = jnp.ones((NW, 1), jnp.float32)
  deg = lax.dot_general(
      deg_ref[...], ones_w, (((0,), (0,)), ((), ())),
      preferred_element_type=jnp.float32) + 1.0
  dinv = lax.rsqrt(deg)
  xw = jnp.dot(x_ref[...], w_ref[...], preferred_element_type=jnp.float32)
  y_ref[...] = xw * dinv
  dinv_ref[...] = dinv


def _tc1(deg_p, x, w1):
  return pl.pallas_call(
      _tc1_body,
      grid=(NPAD // _RB,),
      in_specs=[
          pl.BlockSpec((NW, _RB), lambda i: (0, i)),
          pl.BlockSpec((_RB, D), lambda i: (i, 0)),
          pl.BlockSpec((D, D), lambda i: (0, 0)),
      ],
      out_specs=[
          pl.BlockSpec((_RB, D), lambda i: (i, 0)),
          pl.BlockSpec((_RB, 1), lambda i: (i, 0)),
      ],
      out_shape=[
          jax.ShapeDtypeStruct((NPAD, D), jnp.float32),
          jax.ShapeDtypeStruct((NPAD, 1), jnp.float32),
      ],
  )(deg_p, x, w1)


def _tc2_body(agg_ref, y1_ref, dinv_ref, b1_ref, w2_ref, y2_ref):
  tot = agg_ref[0] + agg_ref[1] + y1_ref[...]
  h = jnp.maximum(tot * dinv_ref[...] + b1_ref[...], 0.0)
  y2_ref[...] = jnp.dot(
      h, w2_ref[...], preferred_element_type=jnp.float32) * dinv_ref[...]


def _tc2(agg1, y1, dinv, b1, w2):
  return pl.pallas_call(
      _tc2_body,
      grid=(NPAD // _RB,),
      in_specs=[
          pl.BlockSpec((NC, _RB, D), lambda i: (0, i, 0)),
          pl.BlockSpec((_RB, D), lambda i: (i, 0)),
          pl.BlockSpec((_RB, 1), lambda i: (i, 0)),
          pl.BlockSpec((1, D), lambda i: (0, 0)),
          pl.BlockSpec((D, D), lambda i: (0, 0)),
      ],
      out_specs=pl.BlockSpec((_RB, D), lambda i: (i, 0)),
      out_shape=jax.ShapeDtypeStruct((NPAD, D), jnp.float32),
  )(agg1, y1, dinv, b1, w2)


def _tc3_body(agg_ref, y2_ref, dinv_ref, b2_ref, h2_ref):
  tot = agg_ref[0] + agg_ref[1] + y2_ref[...]
  h2_ref[...] = tot * dinv_ref[...] + b2_ref[...]


def _tc3(agg2, y2, dinv, b2):
  return pl.pallas_call(
      _tc3_body,
      grid=(NPAD // _RB,),
      in_specs=[
          pl.BlockSpec((NC, _RB, D), lambda i: (0, i, 0)),
          pl.BlockSpec((_RB, D), lambda i: (i, 0)),
          pl.BlockSpec((_RB, 1), lambda i: (i, 0)),
          pl.BlockSpec((1, D), lambda i: (0, 0)),
      ],
      out_specs=pl.BlockSpec((_RB, D), lambda i: (i, 0)),
      out_shape=jax.ShapeDtypeStruct((NPAD, D), jnp.float32),
  )(agg2, y2, dinv, b2)


def kernel(Features, A, E, W1, b1, W2, b2):
  srcE = A[0].reshape(NC, NS, E_CH * E_CE)
  dstE = A[1].reshape(NC, NS, E_CH * E_CE)
  dstW = A[1].reshape(NW, 1, EW)
  epad = jnp.concatenate(
      [E, jnp.zeros((2, P_PAD - NPRED), dtype=E.dtype)], axis=1)
  e0 = epad[0].reshape(NC, NS, P_CH, P_CP)
  e1 = epad[1].reshape(NC, NS, P_CH, P_CP)
  z128 = jnp.zeros((128, D), jnp.float32)
  b1r = b1.reshape(1, D)
  b2r = b2.reshape(1, D)

  deg_p = _sc_degree(dstW).reshape(NW, NPAD)
  y1, dinv = _tc1(deg_p, Features, W1)
  agg1 = _sc_agg(y1, srcE, dstE, z128)
  y2 = _tc2(agg1, y1, dinv, b1r, W2)
  agg2 = _sc_agg(y2, srcE, dstE, z128)
  h2 = _tc3(agg2, y2, dinv, b2r)
  scores = _sc_score(h2, e0, e1)
  return scores[:NPRED]
